# baseline (speedup 1.0000x reference)
import functools

import jax
import jax.numpy as jnp
from jax import lax
from jax.experimental import pallas as pl
from jax.experimental.pallas import tpu as pltpu
from jax.experimental.pallas import tpu_sc as plsc

_H = 512
_W = 512
_C = 192
_C_BLK = 12
_L = 16
_TILES = 16
_ROWS_PER_TILE = _H // _TILES

_HB = 282927299
_LB = 2961979927
_R_INT = 37


def _prng_consts():
    return (
        _HB >> 31, _HB & 0x7FFFFFFF, _LB >> 31, _LB & 0x7FFFFFFF,
        _R_INT * _R_INT,
    )


def _randint_from_count(n, hb_top, hb31, lb_top, lb31):
    span = jnp.where(n <= 0, 1, n)

    def mulmod(a, b):
        hi = (((a * (b // 512)) % span) * 512) % span
        lo = (a * (b % 512)) % span
        return (hi + lo) % span

    m16 = 65536 % span
    w32 = mulmod(m16, m16)
    p31 = mulmod(m16, 32768 % span)
    hbm = (hb_top * p31 + hb31 % span) % span
    lbm = (lb_top * p31 + lb31 % span) % span

    def u32mod(s):
        base = (s % span + span) % span
        return (base + jnp.where(s < 0, w32, 0)) % span

    mult = u32mod(m16 * m16)
    return u32mod(hbm * mult + lbm)


def _make_sc_select(consts):
    hb_top, hb31, lb_top, lb31, _r2 = consts
    mesh = plsc.VectorSubcoreMesh(core_axis_name="c", subcore_axis_name="s")

    @functools.partial(
        pl.kernel,
        out_type=jax.ShapeDtypeStruct((_L,), jnp.int32),
        mesh=mesh,
        compiler_params=pltpu.CompilerParams(needs_layout_passes=False),
        scratch_types=[
            pltpu.VMEM((_ROWS_PER_TILE, _W), jnp.float32),
            pltpu.VMEM((_ROWS_PER_TILE,), jnp.int32),
            pltpu.VMEM_SHARED((_H,), jnp.int32),
            pltpu.VMEM((_H,), jnp.int32),
            pltpu.VMEM((1, _W), jnp.float32),
            pltpu.VMEM((_L,), jnp.int32),
        ],
    )
    def sc_select(label_hbm, sel_hbm, buf_v, cnt_v, shared_cnt, cnts_v,
                  row_v, sel_v):
        cid = lax.axis_index("c")
        sid = lax.axis_index("s")
        lanes = lax.iota(jnp.int32, _L)
        chunks_per_row = _W // _L

        @pl.when(cid == 0)
        def _phase_a():
            base_row = sid * _ROWS_PER_TILE
            pltpu.sync_copy(
                label_hbm.at[0, pl.ds(base_row, _ROWS_PER_TILE), :], buf_v)

            def count_row(r, _):
                parts = [
                    jnp.where(buf_v[r, pl.ds(j * _L, _L)] > 0.5, 1, 0)
                    .astype(jnp.int32)
                    for j in range(chunks_per_row)
                ]
                while len(parts) > 1:
                    parts = [a + b for a, b in zip(parts[::2], parts[1::2])]
                rowsum = jnp.sum(parts[0])
                for g in range(_ROWS_PER_TILE // _L):
                    seg = cnt_v[pl.ds(g * _L, _L)]
                    cnt_v[pl.ds(g * _L, _L)] = jnp.where(
                        lanes == r - g * _L, rowsum, seg)
                return 0

            lax.fori_loop(0, _ROWS_PER_TILE, count_row, 0)
            pltpu.sync_copy(
                cnt_v, shared_cnt.at[pl.ds(base_row, _ROWS_PER_TILE)])
            plsc.subcore_barrier()

            @pl.when(sid == 0)
            def _phase_b():
                pltpu.sync_copy(shared_cnt, cnts_v)
                nchunks = _H // _L

                parts = [cnts_v[pl.ds(k * _L, _L)] for k in range(nchunks)]
                while len(parts) > 1:
                    parts = [a + b for a, b in zip(parts[::2], parts[1::2])]
                n = jnp.sum(parts[0])

                i = _randint_from_count(n, hb_top, hb31, lb_top, lb31)
                target = i + 1

                cum = jnp.int32(0)
                y0f = jnp.int32(_H)
                prev = jnp.int32(0)
                for k in range(nchunks):
                    c = cnts_v[pl.ds(k * _L, _L)]
                    pc = plsc.cumsum(c) + cum
                    hit = pc >= target
                    cand = jnp.min(jnp.where(hit, lanes + k * _L, _H))
                    newly = (y0f == _H) & (cand < _H)
                    prev_cand = jnp.sum(
                        jnp.where(lanes + k * _L == cand, pc - c, 0))
                    prev = jnp.where(newly, prev_cand, prev)
                    y0f = jnp.minimum(y0f, cand)
                    cum = cum + jnp.sum(c)
                y0 = jnp.where(n > 0, y0f, 0)
                tin = target - prev

                pltpu.sync_copy(label_hbm.at[0, pl.ds(y0, 1), :], row_v)

                cum2 = jnp.int32(0)
                x0f = jnp.int32(_W)
                for k in range(chunks_per_row):
                    v = row_v[0, pl.ds(k * _L, _L)]
                    m = v > 0.5
                    mi = jnp.where(m, 1, 0).astype(jnp.int32)
                    pc = plsc.cumsum(mi) + cum2
                    hit = m & (pc == tin)
                    cand = jnp.min(jnp.where(hit, lanes + k * _L, _W))
                    x0f = jnp.minimum(x0f, cand)
                    cum2 = cum2 + jnp.sum(mi)
                x0 = jnp.where(n > 0, x0f, 0)

                sel_v[...] = jnp.where(
                    lanes == 0, y0, jnp.where(lanes == 1, x0, 0))
                pltpu.sync_copy(sel_v, sel_hbm)

    return sc_select


def _make_tc_erase(consts):
    _hb_top, _hb31, _lb_top, _lb31, r2 = consts

    def tc_erase(sel_ref, img_ref, out_ref, mask_ref):
        @pl.when(pl.program_id(0) == 0)
        def _():
            y0 = sel_ref[0]
            x0 = sel_ref[1]
            yy = lax.broadcasted_iota(jnp.int32, (_H, _W), 0)
            xx = lax.broadcasted_iota(jnp.int32, (_H, _W), 1)
            d2 = (yy - y0) * (yy - y0) + (xx - x0) * (xx - x0)
            mask_ref[...] = jnp.where(d2 <= r2, 0.0, 1.0).astype(jnp.float32)

        out_ref[...] = img_ref[...] * mask_ref[...][None, :, :]

    return tc_erase


def kernel(img, label):
    consts = _prng_consts()
    sel = _make_sc_select(consts)(label)

    erased = pl.pallas_call(
        _make_tc_erase(consts),
        grid=(_C // _C_BLK,),
        out_shape=jax.ShapeDtypeStruct((_C, _H, _W), jnp.float32),
        in_specs=[
            pl.BlockSpec(memory_space=pltpu.SMEM),
            pl.BlockSpec((_C_BLK, _H, _W), lambda c: (c, 0, 0)),
        ],
        out_specs=pl.BlockSpec((_C_BLK, _H, _W), lambda c: (c, 0, 0)),
        scratch_shapes=[pltpu.VMEM((_H, _W), jnp.float32)],
    )(sel, img)

    return (erased, label)


# speedup vs baseline: 1.0164x; 1.0164x over previous
import functools

import jax
import jax.numpy as jnp
from jax import lax
from jax.experimental import pallas as pl
from jax.experimental.pallas import tpu as pltpu
from jax.experimental.pallas import tpu_sc as plsc

_H = 512
_W = 512
_C = 192
_C_BLK = 12
_L = 16
_TILES = 16
_ROWS_PER_TILE = _H // _TILES

_HB = 282927299
_LB = 2961979927
_R_INT = 37


def _prng_consts():
    return (
        _HB >> 31, _HB & 0x7FFFFFFF, _LB >> 31, _LB & 0x7FFFFFFF,
        _R_INT * _R_INT,
    )


def _randint_from_count(n, hb_top, hb31, lb_top, lb31):
    span = jnp.where(n <= 0, 1, n)

    def mulmod(a, b):
        hi = (((a * (b // 512)) % span) * 512) % span
        lo = (a * (b % 512)) % span
        return (hi + lo) % span

    m16 = 65536 % span
    w32 = mulmod(m16, m16)
    p31 = mulmod(m16, 32768 % span)
    hbm = (hb_top * p31 + hb31 % span) % span
    lbm = (lb_top * p31 + lb31 % span) % span

    def u32mod(s):
        base = (s % span + span) % span
        return (base + jnp.where(s < 0, w32, 0)) % span

    mult = u32mod(m16 * m16)
    return u32mod(hbm * mult + lbm)


def _make_sc_select(consts):
    hb_top, hb31, lb_top, lb31, _r2 = consts
    mesh = plsc.VectorSubcoreMesh(core_axis_name="c", subcore_axis_name="s")

    @functools.partial(
        pl.kernel,
        out_type=jax.ShapeDtypeStruct((_L,), jnp.int32),
        mesh=mesh,
        compiler_params=pltpu.CompilerParams(needs_layout_passes=False),
        scratch_types=[
            pltpu.VMEM((_ROWS_PER_TILE, _W), jnp.float32),
            pltpu.VMEM((_ROWS_PER_TILE,), jnp.int32),
            pltpu.VMEM_SHARED((_H,), jnp.int32),
            pltpu.VMEM((_H,), jnp.int32),
            pltpu.VMEM((1, _W), jnp.float32),
            pltpu.VMEM((_L,), jnp.int32),
        ],
    )
    def sc_select(label_hbm, sel_hbm, buf_v, cnt_v, shared_cnt, cnts_v,
                  row_v, sel_v):
        cid = lax.axis_index("c")
        sid = lax.axis_index("s")
        lanes = lax.iota(jnp.int32, _L)
        chunks_per_row = _W // _L

        @pl.when(cid == 0)
        def _phase_a():
            base_row = sid * _ROWS_PER_TILE
            pltpu.sync_copy(
                label_hbm.at[0, pl.ds(base_row, _ROWS_PER_TILE), :], buf_v)

            def count_row(r, _):
                parts = [
                    jnp.where(buf_v[r, pl.ds(j * _L, _L)] > 0.5, 1, 0)
                    .astype(jnp.int32)
                    for j in range(chunks_per_row)
                ]
                while len(parts) > 1:
                    parts = [a + b for a, b in zip(parts[::2], parts[1::2])]
                rowsum = jnp.sum(parts[0])
                for g in range(_ROWS_PER_TILE // _L):
                    seg = cnt_v[pl.ds(g * _L, _L)]
                    cnt_v[pl.ds(g * _L, _L)] = jnp.where(
                        lanes == r - g * _L, rowsum, seg)
                return 0

            lax.fori_loop(0, _ROWS_PER_TILE, count_row, 0)
            pltpu.sync_copy(
                cnt_v, shared_cnt.at[pl.ds(base_row, _ROWS_PER_TILE)])
            plsc.subcore_barrier()

            @pl.when(sid == 0)
            def _phase_b():
                pltpu.sync_copy(shared_cnt, cnts_v)
                nchunks = _H // _L

                def add_chunk(k, acc):
                    return acc + cnts_v[pl.ds(k * _L, _L)]

                n = jnp.sum(lax.fori_loop(
                    0, nchunks, add_chunk, jnp.zeros((_L,), jnp.int32)))

                i = _randint_from_count(n, hb_top, hb31, lb_top, lb31)
                target = i + 1

                def find_row(k, carry):
                    cum, y0, prev = carry
                    c = cnts_v[pl.ds(k * _L, _L)]
                    pc = plsc.cumsum(c) + cum
                    hit = pc >= target
                    cand = jnp.min(jnp.where(hit, lanes + k * _L, _H))
                    newly = (y0 == _H) & (cand < _H)
                    prev_cand = jnp.sum(
                        jnp.where(lanes + k * _L == cand, pc - c, 0))
                    return (cum + jnp.sum(c), jnp.minimum(y0, cand),
                            jnp.where(newly, prev_cand, prev))

                _, y0f, prev = lax.fori_loop(
                    0, nchunks, find_row,
                    (jnp.int32(0), jnp.int32(_H), jnp.int32(0)))
                y0 = jnp.where(n > 0, y0f, 0)
                tin = target - prev

                pltpu.sync_copy(label_hbm.at[0, pl.ds(y0, 1), :], row_v)

                def find_col(k, carry):
                    cum, x0 = carry
                    v = row_v[0, pl.ds(k * _L, _L)]
                    m = v > 0.5
                    mi = jnp.where(m, 1, 0).astype(jnp.int32)
                    pc = plsc.cumsum(mi) + cum
                    hit = m & (pc == tin)
                    cand = jnp.min(jnp.where(hit, lanes + k * _L, _W))
                    return (cum + jnp.sum(mi), jnp.minimum(x0, cand))

                _, x0f = lax.fori_loop(0, chunks_per_row, find_col,
                                       (jnp.int32(0), jnp.int32(_W)))
                x0 = jnp.where(n > 0, x0f, 0)

                sel_v[...] = jnp.where(
                    lanes == 0, y0, jnp.where(lanes == 1, x0, 0))
                pltpu.sync_copy(sel_v, sel_hbm)

    return sc_select


def _make_tc_erase(consts):
    _hb_top, _hb31, _lb_top, _lb31, r2 = consts

    def tc_erase(sel_ref, img_ref, out_ref, mask_ref):
        @pl.when(pl.program_id(0) == 0)
        def _():
            y0 = sel_ref[0]
            x0 = sel_ref[1]
            yy = lax.broadcasted_iota(jnp.int32, (_H, _W), 0)
            xx = lax.broadcasted_iota(jnp.int32, (_H, _W), 1)
            d2 = (yy - y0) * (yy - y0) + (xx - x0) * (xx - x0)
            mask_ref[...] = jnp.where(d2 <= r2, 0.0, 1.0).astype(jnp.float32)

        out_ref[...] = img_ref[...] * mask_ref[...][None, :, :]

    return tc_erase


def kernel(img, label):
    consts = _prng_consts()
    sel = _make_sc_select(consts)(label)

    erased = pl.pallas_call(
        _make_tc_erase(consts),
        grid=(_C // _C_BLK,),
        out_shape=jax.ShapeDtypeStruct((_C, _H, _W), jnp.float32),
        in_specs=[
            pl.BlockSpec(memory_space=pltpu.SMEM),
            pl.BlockSpec((_C_BLK, _H, _W), lambda c: (c, 0, 0)),
        ],
        out_specs=pl.BlockSpec((_C_BLK, _H, _W), lambda c: (c, 0, 0)),
        scratch_shapes=[pltpu.VMEM((_H, _W), jnp.float32)],
    )(sel, img)

    return (erased, label)


# speedup vs baseline: 1.0244x; 1.0079x over previous
import functools

import jax
import jax.numpy as jnp
from jax import lax
from jax.experimental import pallas as pl
from jax.experimental.pallas import tpu as pltpu
from jax.experimental.pallas import tpu_sc as plsc

_H = 512
_W = 512
_C = 192
_C_BLK = 14
_L = 16
_TILES = 16
_ROWS_PER_TILE = _H // _TILES

_HB = 282927299
_LB = 2961979927
_R_INT = 37


def _prng_consts():
    return (
        _HB >> 31, _HB & 0x7FFFFFFF, _LB >> 31, _LB & 0x7FFFFFFF,
        _R_INT * _R_INT,
    )


def _randint_from_count(n, hb_top, hb31, lb_top, lb31):
    span = jnp.where(n <= 0, 1, n)

    def mulmod(a, b):
        hi = (((a * (b // 512)) % span) * 512) % span
        lo = (a * (b % 512)) % span
        return (hi + lo) % span

    m16 = 65536 % span
    w32 = mulmod(m16, m16)
    p31 = mulmod(m16, 32768 % span)
    hbm = (hb_top * p31 + hb31 % span) % span
    lbm = (lb_top * p31 + lb31 % span) % span

    def u32mod(s):
        base = (s % span + span) % span
        return (base + jnp.where(s < 0, w32, 0)) % span

    mult = u32mod(m16 * m16)
    return u32mod(hbm * mult + lbm)


def _make_sc_select(consts):
    hb_top, hb31, lb_top, lb31, _r2 = consts
    mesh = plsc.VectorSubcoreMesh(core_axis_name="c", subcore_axis_name="s")

    @functools.partial(
        pl.kernel,
        out_type=jax.ShapeDtypeStruct((_L,), jnp.int32),
        mesh=mesh,
        compiler_params=pltpu.CompilerParams(needs_layout_passes=False),
        scratch_types=[
            pltpu.VMEM((_ROWS_PER_TILE, _W), jnp.float32),
            pltpu.VMEM((_ROWS_PER_TILE,), jnp.int32),
            pltpu.VMEM_SHARED((_H,), jnp.int32),
            pltpu.VMEM((_H,), jnp.int32),
            pltpu.VMEM((1, _W), jnp.float32),
            pltpu.VMEM((_L,), jnp.int32),
        ],
    )
    def sc_select(label_hbm, sel_hbm, buf_v, cnt_v, shared_cnt, cnts_v,
                  row_v, sel_v):
        cid = lax.axis_index("c")
        sid = lax.axis_index("s")
        lanes = lax.iota(jnp.int32, _L)
        chunks_per_row = _W // _L

        @pl.when(cid == 0)
        def _phase_a():
            base_row = sid * _ROWS_PER_TILE
            pltpu.sync_copy(
                label_hbm.at[0, pl.ds(base_row, _ROWS_PER_TILE), :], buf_v)

            def count_row(r, _):
                parts = [
                    jnp.where(buf_v[r, pl.ds(j * _L, _L)] > 0.5, 1, 0)
                    .astype(jnp.int32)
                    for j in range(chunks_per_row)
                ]
                while len(parts) > 1:
                    parts = [a + b for a, b in zip(parts[::2], parts[1::2])]
                rowsum = jnp.sum(parts[0])
                for g in range(_ROWS_PER_TILE // _L):
                    seg = cnt_v[pl.ds(g * _L, _L)]
                    cnt_v[pl.ds(g * _L, _L)] = jnp.where(
                        lanes == r - g * _L, rowsum, seg)
                return 0

            lax.fori_loop(0, _ROWS_PER_TILE, count_row, 0)
            pltpu.sync_copy(
                cnt_v, shared_cnt.at[pl.ds(base_row, _ROWS_PER_TILE)])
            plsc.subcore_barrier()

            @pl.when(sid == 0)
            def _phase_b():
                pltpu.sync_copy(shared_cnt, cnts_v)
                nchunks = _H // _L

                def add_chunk(k, acc):
                    return acc + cnts_v[pl.ds(k * _L, _L)]

                n = jnp.sum(lax.fori_loop(
                    0, nchunks, add_chunk, jnp.zeros((_L,), jnp.int32)))

                i = _randint_from_count(n, hb_top, hb31, lb_top, lb31)
                target = i + 1

                def find_row(k, carry):
                    cum, y0, prev = carry
                    c = cnts_v[pl.ds(k * _L, _L)]
                    pc = plsc.cumsum(c) + cum
                    hit = pc >= target
                    cand = jnp.min(jnp.where(hit, lanes + k * _L, _H))
                    newly = (y0 == _H) & (cand < _H)
                    prev_cand = jnp.sum(
                        jnp.where(lanes + k * _L == cand, pc - c, 0))
                    return (cum + jnp.sum(c), jnp.minimum(y0, cand),
                            jnp.where(newly, prev_cand, prev))

                _, y0f, prev = lax.fori_loop(
                    0, nchunks, find_row,
                    (jnp.int32(0), jnp.int32(_H), jnp.int32(0)))
                y0 = jnp.where(n > 0, y0f, 0)
                tin = target - prev

                pltpu.sync_copy(label_hbm.at[0, pl.ds(y0, 1), :], row_v)

                def find_col(k, carry):
                    cum, x0 = carry
                    v = row_v[0, pl.ds(k * _L, _L)]
                    m = v > 0.5
                    mi = jnp.where(m, 1, 0).astype(jnp.int32)
                    pc = plsc.cumsum(mi) + cum
                    hit = m & (pc == tin)
                    cand = jnp.min(jnp.where(hit, lanes + k * _L, _W))
                    return (cum + jnp.sum(mi), jnp.minimum(x0, cand))

                _, x0f = lax.fori_loop(0, chunks_per_row, find_col,
                                       (jnp.int32(0), jnp.int32(_W)))
                x0 = jnp.where(n > 0, x0f, 0)

                sel_v[...] = jnp.where(
                    lanes == 0, y0, jnp.where(lanes == 1, x0, 0))
                pltpu.sync_copy(sel_v, sel_hbm)

    return sc_select


def _make_tc_erase(consts):
    _hb_top, _hb31, _lb_top, _lb31, r2 = consts

    def tc_erase(sel_ref, img_ref, out_ref, mask_ref):
        @pl.when(pl.program_id(0) == 0)
        def _():
            y0 = sel_ref[0]
            x0 = sel_ref[1]
            yy = lax.broadcasted_iota(jnp.int32, (_H, _W), 0)
            xx = lax.broadcasted_iota(jnp.int32, (_H, _W), 1)
            d2 = (yy - y0) * (yy - y0) + (xx - x0) * (xx - x0)
            mask_ref[...] = jnp.where(d2 <= r2, 0.0, 1.0).astype(jnp.float32)

        out_ref[...] = img_ref[...] * mask_ref[...][None, :, :]

    return tc_erase


def kernel(img, label):
    consts = _prng_consts()
    sel = _make_sc_select(consts)(label)

    erased = pl.pallas_call(
        _make_tc_erase(consts),
        grid=(-(-_C // _C_BLK),),
        out_shape=jax.ShapeDtypeStruct((_C, _H, _W), jnp.float32),
        in_specs=[
            pl.BlockSpec(memory_space=pltpu.SMEM),
            pl.BlockSpec((_C_BLK, _H, _W), lambda c: (c, 0, 0)),
        ],
        out_specs=pl.BlockSpec((_C_BLK, _H, _W), lambda c: (c, 0, 0)),
        scratch_shapes=[pltpu.VMEM((_H, _W), jnp.float32)],
    )(sel, img)

    return (erased, label)
